# baseline (speedup 1.0000x reference)
import functools

import jax
import jax.numpy as jnp
from jax import lax
from jax.experimental import pallas as pl
from jax.experimental.pallas import tpu as pltpu
from jax.experimental.pallas import tpu_sc as plsc

B = 16384
D = 256
K = 8192
DECAY = 0.99
EPS = 1e-05

BM = 256
KC = 1024
NB = B // BM

NC = 2
NS = 16
CHUNK = 128


def _rtne_bf16(x):
    b = jax.lax.bitcast_convert_type(x, jnp.int32)
    r = (b + jnp.int32(0x7FFF) + ((b >> 16) & jnp.int32(1))) & jnp.int32(
        -65536)
    return jax.lax.bitcast_convert_type(r, jnp.float32)


def _k2_body(z_ref, w_ref, codes_ref, zn_ref, en_ref):
    i = pl.program_id(0)

    @pl.when(i == 0)
    def _():
        w = w_ref[...]
        nw = jnp.sqrt(jnp.sum(w * w, axis=1, keepdims=True))
        en = w / jnp.maximum(nw, EPS)
        en_ref[...] = _rtne_bf16(en)

    z = z_ref[...]
    nz = jnp.sqrt(jnp.sum(z * z, axis=1, keepdims=True))
    zn = z / jnp.maximum(nz, EPS)
    zn_ref[...] = zn
    znb = _rtne_bf16(zn).astype(jnp.bfloat16)

    best = jnp.full((BM, 1), -jnp.inf, dtype=jnp.float32)
    bestid = jnp.zeros((BM, 1), dtype=jnp.int32)
    for c in range(K // KC):
        en_c = en_ref[pl.ds(c * KC, KC), :].astype(jnp.bfloat16)
        sim = lax.dot_general(znb, en_c, (((1,), (1,)), ((), ())),
                              preferred_element_type=jnp.float32)
        cm = jnp.max(sim, axis=1, keepdims=True)
        ids = jax.lax.broadcasted_iota(jnp.int32, (BM, KC), 1) + c * KC
        cid = jnp.min(jnp.where(sim == cm, ids, K), axis=1, keepdims=True)
        upd = cm > best
        bestid = jnp.where(upd, cid, bestid)
        best = jnp.maximum(best, cm)
    codes_ref[...] = bestid


def _k2(z_e, weight):
    return pl.pallas_call(
        _k2_body,
        grid=(NB,),
        in_specs=[
            pl.BlockSpec((BM, D), lambda i: (i, 0)),
            pl.BlockSpec((K, D), lambda i: (0, 0)),
        ],
        out_specs=[
            pl.BlockSpec((BM, 1), lambda i: (i, 0)),
            pl.BlockSpec((BM, D), lambda i: (i, 0)),
        ],
        out_shape=[
            jax.ShapeDtypeStruct((B, 1), jnp.int32),
            jax.ShapeDtypeStruct((B, D), jnp.float32),
        ],
        scratch_shapes=[pltpu.VMEM((K, D), jnp.float32)],
    )(z_e, weight)


SHARD = K // (NC * NS)
CSTAGE = 1024
GROWS = 32


def _k3_body(codes_hbm, zn_hbm, zeros_hbm, zeros16_hbm, dw_hbm, cnt_hbm,
             cbuf, plist, gidx, gbuf, acc, cacc, sem):
    c = lax.axis_index("c")
    s = lax.axis_index("s")
    t = c * NS + s
    lo = t * SHARD

    pltpu.sync_copy(zeros_hbm, acc.at[pl.ds(0, SHARD)])
    pltpu.sync_copy(zeros16_hbm, cacc.at[pl.ds(0, SHARD)])

    zv = jnp.zeros((16,), jnp.int32)

    def stage(cs, ptr):
        pltpu.sync_copy(codes_hbm.at[pl.ds(cs * CSTAGE, CSTAGE)],
                        cbuf.at[pl.ds(0, CSTAGE)])

        def lane(l, pp):
            cv = cbuf[pl.ds(l, 16)][0]
            inr = (cv >= lo) & (cv < lo + SHARD)
            plist[pl.ds(pp, 16)] = zv + ((cs * CSTAGE + l) * 512 + (cv - lo))
            return pp + inr.astype(jnp.int32)

        return lax.fori_loop(0, CSTAGE, lane, ptr)

    nmine = lax.fori_loop(0, B // CSTAGE, stage, jnp.int32(0))

    for k in range(GROWS // 16):
        plist[pl.ds(nmine + k * 16, 16)] = zv + SHARD

    ones16 = jnp.ones((16,), jnp.float32)

    def bchunk(ch, carry):
        base = ch * GROWS

        @pl.when(base < nmine)
        def _():
            for k in range(GROWS // 16):
                gidx[pl.ds(k * 16, 16)] = (
                    plist[pl.ds(base + k * 16, 16)] >> 9)
            pltpu.async_copy(zn_hbm.at[gidx], gbuf, sem).wait()

            def row(j, c2):
                pk = plist[pl.ds(base + j, 16)][0]
                cl = jnp.where(base + j < nmine, pk & 511, SHARD)
                for v in range(D // 16):
                    plsc.addupdate(acc.at[cl, pl.ds(v * 16, 16)],
                                   gbuf[j, pl.ds(v * 16, 16)])
                plsc.addupdate(cacc.at[cl, pl.ds(0, 16)], ones16)
                return c2

            lax.fori_loop(0, GROWS, row, 0)

        return carry

    lax.fori_loop(0, B // GROWS, bchunk, 0)

    pltpu.sync_copy(acc.at[pl.ds(0, SHARD)], dw_hbm.at[pl.ds(lo, SHARD)])
    pltpu.sync_copy(cacc.at[pl.ds(0, SHARD)], cnt_hbm.at[pl.ds(lo, SHARD)])


def _k3(codes, zn, zeros_in, zeros16_in):
    mesh = plsc.VectorSubcoreMesh(core_axis_name="c", subcore_axis_name="s",
                                  num_cores=NC, num_subcores=NS)
    kfn = pl.kernel(
        _k3_body,
        out_type=[
            jax.ShapeDtypeStruct((K, D), jnp.float32),
            jax.ShapeDtypeStruct((K, 16), jnp.float32),
        ],
        mesh=mesh,
        scratch_types=[
            pltpu.VMEM((CSTAGE + 16,), jnp.int32),
            pltpu.VMEM((B + GROWS + 16,), jnp.int32),
            pltpu.VMEM((GROWS,), jnp.int32),
            pltpu.VMEM((GROWS, D), jnp.float32),
            pltpu.VMEM((SHARD + 8, D), jnp.float32),
            pltpu.VMEM((SHARD + 8, 16), jnp.float32),
            pltpu.SemaphoreType.DMA,
        ],
    )
    return kfn(codes, zn, zeros_in, zeros16_in)


def _k4_body(cnt_ref, ecs_ref, dw_ref, emaw_ref, nw_ref):
    counts = cnt_ref[:, pl.ds(0, 1)]
    new_cs = ecs_ref[...] * DECAY + counts * (1.0 - DECAY)
    n = jnp.sum(new_cs)
    cluster_size = (new_cs + EPS) / (n + float(K) * EPS) * n
    new_ema_w = emaw_ref[...] * DECAY + dw_ref[...] * (1.0 - DECAY)
    em = new_ema_w / jnp.maximum(cluster_size, EPS)
    nrm = jnp.sqrt(jnp.sum(em * em, axis=1, keepdims=True))
    nw_ref[...] = em / jnp.maximum(nrm, EPS)


def _k4(cnt, ema_cs2d, dw, ema_w):
    return pl.pallas_call(
        _k4_body,
        out_shape=jax.ShapeDtypeStruct((K, D), jnp.float32),
    )(cnt, ema_cs2d, dw, ema_w)


def _k5_body(table_hbm, codes_hbm, out_hbm, idx_v, rows_v, sem):
    c = lax.axis_index("c")
    s = lax.axis_index("s")
    wid = s * NC + c
    per_w = B // (NC * NS)

    def chunk(ch, carry):
        base = wid * per_w + ch * CHUNK
        pltpu.sync_copy(codes_hbm.at[pl.ds(base, CHUNK)], idx_v)
        pltpu.async_copy(table_hbm.at[idx_v], rows_v, sem).wait()
        pltpu.sync_copy(rows_v, out_hbm.at[pl.ds(base, CHUNK)])
        return carry

    lax.fori_loop(0, per_w // CHUNK, chunk, 0)


def _k5(new_weight, codes):
    mesh = plsc.VectorSubcoreMesh(core_axis_name="c", subcore_axis_name="s",
                                  num_cores=NC, num_subcores=NS)
    kfn = pl.kernel(
        _k5_body,
        out_type=jax.ShapeDtypeStruct((B, D), jnp.float32),
        mesh=mesh,
        scratch_types=[
            pltpu.VMEM((CHUNK,), jnp.int32),
            pltpu.VMEM((CHUNK, D), jnp.float32),
            pltpu.SemaphoreType.DMA,
        ],
    )
    return kfn(new_weight, codes)


def _k6_body(z_ref, es_ref, zq_ref, d_ref):
    z = z_ref[...]
    nz = jnp.sqrt(jnp.sum(z * z, axis=1, keepdims=True))
    zn = z / jnp.maximum(nz, EPS)
    es = es_ref[...]
    ne = jnp.sqrt(jnp.sum(es * es, axis=1, keepdims=True))
    esn = es / jnp.maximum(ne, EPS)
    z_q = es * jnp.maximum(nz, EPS)
    zq_ref[...] = z + (z_q - z)
    d_ref[...] = 1.0 - jnp.sum(zn * esn, axis=1, keepdims=True)


def _k6(z_e, e_sel):
    return pl.pallas_call(
        _k6_body,
        grid=(NB,),
        in_specs=[
            pl.BlockSpec((BM, D), lambda i: (i, 0)),
            pl.BlockSpec((BM, D), lambda i: (i, 0)),
        ],
        out_specs=[
            pl.BlockSpec((BM, D), lambda i: (i, 0)),
            pl.BlockSpec((BM, 1), lambda i: (i, 0)),
        ],
        out_shape=[
            jax.ShapeDtypeStruct((B, D), jnp.float32),
            jax.ShapeDtypeStruct((B, 1), jnp.float32),
        ],
    )(z_e, e_sel)


def kernel(z_e, weight, ema_cluster_size, ema_w):
    codes2d, zn = _k2(z_e, weight)
    codes = codes2d.reshape(B)
    zeros_in = jnp.zeros((SHARD, D), jnp.float32)
    zeros16_in = jnp.zeros((SHARD, 16), jnp.float32)
    dw, cnt = _k3(codes, zn, zeros_in, zeros16_in)
    new_weight = _k4(cnt, ema_cluster_size.reshape(K, 1), dw, ema_w)
    e_sel = _k5(new_weight, codes)
    z_q_st, dists2d = _k6(z_e, e_sel)
    return (z_q_st, codes, dists2d.reshape(B))
